# baseline (speedup 1.0000x reference)
import functools

import jax
import jax.numpy as jnp
from jax import lax
from jax.experimental import pallas as pl
from jax.experimental.pallas import tpu as pltpu
from jax.experimental.pallas import tpu_sc as plsc

_LANES = 16


@functools.cache
def _build(V, D, Bt, H):
    try:
        info = plsc.get_sparse_core_info()
        NC, NS = info.num_cores, info.num_subcores
    except Exception:
        NC, NS = 2, 16
    NW = NC * NS
    HT = H // 8
    BT = Bt // 128
    DT = D // 8
    assert H % 8 == 0 and Bt % 128 == 0 and D % 8 == 0
    assert BT % NW == 0
    BW = BT // NW
    n_steps = H
    ROWS = BW * 128

    mesh = plsc.VectorSubcoreMesh(core_axis_name="c", subcore_axis_name="s")

    @functools.partial(
        pl.kernel,
        mesh=mesh,
        compiler_params=pltpu.CompilerParams(
            use_tc_tiling_on_sc=False, needs_layout_passes=False
        ),
        out_type=jax.ShapeDtypeStruct((H, DT, BT * 8, 128), jnp.float32),
        scratch_types=[
            pltpu.VMEM((2, BW, 8, 128), jnp.int32),
            pltpu.VMEM((2, ROWS, D), jnp.float32),
            pltpu.VMEM((2 * DT * BW * 8, 128), jnp.float32),
            pltpu.SemaphoreType.DMA,
            pltpu.SemaphoreType.DMA,
            pltpu.SemaphoreType.DMA,
        ],
    )
    def emb(table_hbm, idx_hbm, out_hbm, idx_v, src_v, dst_v,
            sem_idx, sem_g, sem_o):
        wid = lax.axis_index("s") * NC + lax.axis_index("c")
        bt0 = wid * BW

        def idx_start(ht, islot):
            pltpu.async_copy(
                idx_hbm.at[ht].at[pl.ds(bt0, BW)], idx_v.at[islot], sem_idx
            )

        def idx_wait(islot):
            pltpu.make_async_copy(
                idx_hbm.at[0].at[pl.ds(bt0, BW)], idx_v.at[islot], sem_idx
            ).wait()

        def fire_gathers(step, sslot):
            hs = step % 8
            islot = (step // 8) % 2
            for btj in range(BW):
                pltpu.async_copy(
                    table_hbm.at[idx_v.at[islot].at[btj].at[hs]],
                    src_v.at[sslot].at[pl.ds(btj * 128, 128)],
                    sem_g,
                )

        def drain_gathers(sslot):
            for btj in range(BW):
                pltpu.make_async_copy(
                    table_hbm.at[idx_v.at[0].at[btj].at[0]],
                    src_v.at[sslot].at[pl.ds(btj * 128, 128)],
                    sem_g,
                ).wait()

        def store_start(h, dslot):
            for dt in range(DT):
                pltpu.async_copy(
                    dst_v.at[pl.ds((dslot * DT + dt) * BW * 8, BW * 8)],
                    out_hbm.at[h].at[dt].at[pl.ds(bt0 * 8, BW * 8)],
                    sem_o,
                )

        def store_wait(dslot):
            for dt in range(DT):
                pltpu.make_async_copy(
                    dst_v.at[pl.ds((dslot * DT + dt) * BW * 8, BW * 8)],
                    out_hbm.at[0].at[dt].at[pl.ds(bt0 * 8, BW * 8)],
                    sem_o,
                ).wait()

        lane = jnp.arange(_LANES, dtype=jnp.int32)

        def transpose(b):
            src2 = src_v.at[b]

            @plsc.parallel_loop(0, ROWS // _LANES, unroll=2)
            def _(i):
                rowv = lane + i * _LANES
                btj8 = (i >> 3) * 8
                blk0 = (i & 7) * _LANES
                for d in range(D):
                    colv = jnp.full((_LANES,), d, jnp.int32)
                    vec = plsc.load_gather(src2, [rowv, colv])
                    r = (b * DT + d // 8) * (BW * 8) + (d % 8) + btj8
                    dst_v[r, pl.ds(blk0, _LANES)] = vec

        idx_start(0, 0)
        idx_wait(0)
        fire_gathers(0, 0)

        def outer(o, _):
            for b in range(2):
                step = o * 2 + b

                @pl.when(jnp.logical_and(step % 8 == 0, step + 8 < n_steps))
                def _():
                    idx_start(step // 8 + 1, (step // 8 + 1) % 2)

                drain_gathers(b)

                @pl.when(step + 1 < n_steps)
                def _():
                    @pl.when((step + 1) % 8 == 0)
                    def _():
                        idx_wait(((step + 1) // 8) % 2)

                    fire_gathers(step + 1, 1 - b)

                @pl.when(step >= 2)
                def _():
                    store_wait(b)

                transpose(b)
                store_start(step, b)
            return 0

        lax.fori_loop(0, n_steps // 2, outer, 0)
        for b in range(2):
            store_wait(b)

    return emb


def kernel(y, E):
    Bt, H = y.shape
    V, D = E.shape
    HT, BT, DT = H // 8, Bt // 128, D // 8
    y4 = y.T.reshape(HT, 8, BT, 128).transpose(0, 2, 1, 3)
    o4 = _build(V, D, Bt, H)(E, y4)
    o5 = o4.reshape(H, DT, BT, 8, 128)
    return o5.transpose(2, 4, 0, 1, 3).reshape(Bt, H, D)


# speedup vs baseline: 1.0517x; 1.0517x over previous
import functools

import jax
import jax.numpy as jnp
from jax import lax
from jax.experimental import pallas as pl
from jax.experimental.pallas import tpu as pltpu
from jax.experimental.pallas import tpu_sc as plsc

_LANES = 16


@functools.cache
def _build(V, D, Bt, H):
    try:
        info = plsc.get_sparse_core_info()
        NC, NS = info.num_cores, info.num_subcores
    except Exception:
        NC, NS = 2, 16
    NW = NC * NS
    HT = H // 8
    BT = Bt // 128
    DT = D // 8
    assert H % 8 == 0 and Bt % 128 == 0 and D % 8 == 0
    assert BT % NW == 0
    BW = BT // NW
    n_steps = H
    ROWS = BW * 128

    mesh = plsc.VectorSubcoreMesh(core_axis_name="c", subcore_axis_name="s")

    @functools.partial(
        pl.kernel,
        mesh=mesh,
        compiler_params=pltpu.CompilerParams(
            use_tc_tiling_on_sc=False, needs_layout_passes=False
        ),
        out_type=jax.ShapeDtypeStruct((H, DT, BT * 8, 128), jnp.float32),
        scratch_types=[
            pltpu.VMEM((2, BW, 8, 128), jnp.int32),
            pltpu.VMEM((2, ROWS, D), jnp.float32),
            pltpu.VMEM((2 * DT * BW * 8, 128), jnp.float32),
            pltpu.SemaphoreType.DMA,
            pltpu.SemaphoreType.DMA,
            pltpu.SemaphoreType.DMA,
        ],
    )
    def emb(table_hbm, idx_hbm, out_hbm, idx_v, src_v, dst_v,
            sem_idx, sem_g, sem_o):
        wid = lax.axis_index("s") * NC + lax.axis_index("c")
        bt0 = wid * BW

        def idx_start(ht, islot):
            pltpu.async_copy(
                idx_hbm.at[ht].at[pl.ds(bt0, BW)], idx_v.at[islot], sem_idx
            )

        def idx_wait(islot):
            pltpu.make_async_copy(
                idx_hbm.at[0].at[pl.ds(bt0, BW)], idx_v.at[islot], sem_idx
            ).wait()

        def fire_gathers(step, sslot):
            hs = step % 8
            islot = (step // 8) % 2
            for btj in range(BW):
                pltpu.async_copy(
                    table_hbm.at[idx_v.at[islot].at[btj].at[hs]],
                    src_v.at[sslot].at[pl.ds(btj * 128, 128)],
                    sem_g,
                )

        def drain_gathers(sslot):
            pltpu.make_async_copy(
                table_hbm.at[idx_v.at[0].at[0].at[0]],
                src_v.at[sslot],
                sem_g,
            ).wait()

        def store_start(h, dslot):
            for dt in range(DT):
                pltpu.async_copy(
                    dst_v.at[pl.ds((dslot * DT + dt) * BW * 8, BW * 8)],
                    out_hbm.at[h].at[dt].at[pl.ds(bt0 * 8, BW * 8)],
                    sem_o,
                )

        def store_wait(dslot):
            pltpu.make_async_copy(
                dst_v.at[pl.ds(dslot * DT * BW * 8, DT * BW * 8)],
                out_hbm.at[0].at[0].at[pl.ds(0, DT * BW * 8)],
                sem_o,
            ).wait()

        lane = jnp.arange(_LANES, dtype=jnp.int32)
        r_const = (lane // 8) * (BW * 8) + (lane % 8)

        def transpose(b):
            src = src_v.at[b]

            @plsc.parallel_loop(0, ROWS, unroll=8)
            def _(i):
                btj = i // 128
                rloc = i - btj * 128
                bl_vec = jnp.zeros((_LANES,), jnp.int32) + rloc
                for half in range(2):
                    vec = src[i, pl.ds(half * _LANES, _LANES)]
                    row_vec = r_const + (
                        (b * DT + half * 2) * (BW * 8) + btj * 8
                    )
                    plsc.store_scatter(dst_v, [row_vec, bl_vec], vec)

        idx_start(0, 0)
        idx_wait(0)
        fire_gathers(0, 0)

        def outer(o, _):
            for b in range(2):
                step = o * 2 + b

                @pl.when(jnp.logical_and(step % 8 == 0, step + 8 < n_steps))
                def _():
                    idx_start(step // 8 + 1, (step // 8 + 1) % 2)

                drain_gathers(b)

                @pl.when(step + 1 < n_steps)
                def _():
                    @pl.when((step + 1) % 8 == 0)
                    def _():
                        idx_wait(((step + 1) // 8) % 2)

                    fire_gathers(step + 1, 1 - b)

                @pl.when(step >= 2)
                def _():
                    store_wait(b)

                transpose(b)
                store_start(step, b)
            return 0

        lax.fori_loop(0, n_steps // 2, outer, 0)
        for b in range(2):
            store_wait(b)

    return emb


def kernel(y, E):
    Bt, H = y.shape
    V, D = E.shape
    HT, BT, DT = H // 8, Bt // 128, D // 8
    y4 = y.T.reshape(HT, 8, BT, 128).transpose(0, 2, 1, 3)
    o4 = _build(V, D, Bt, H)(E, y4)
    o5 = o4.reshape(H, DT, BT, 8, 128)
    return o5.transpose(2, 4, 0, 1, 3).reshape(Bt, H, D)


# speedup vs baseline: 1.0531x; 1.0013x over previous
import functools

import jax
import jax.numpy as jnp
from jax import lax
from jax.experimental import pallas as pl
from jax.experimental.pallas import tpu as pltpu
from jax.experimental.pallas import tpu_sc as plsc

_LANES = 16


@functools.cache
def _build(V, D, Bt, H):
    try:
        info = plsc.get_sparse_core_info()
        NC, NS = info.num_cores, info.num_subcores
    except Exception:
        NC, NS = 2, 16
    NW = NC * NS
    HT = H // 8
    BT = Bt // 128
    DT = D // 8
    assert H % 8 == 0 and Bt % 128 == 0 and D % 8 == 0
    assert BT % NW == 0
    BW = BT // NW
    n_steps = H
    ROWS = BW * 128

    mesh = plsc.VectorSubcoreMesh(core_axis_name="c", subcore_axis_name="s")

    @functools.partial(
        pl.kernel,
        mesh=mesh,
        compiler_params=pltpu.CompilerParams(
            use_tc_tiling_on_sc=False, needs_layout_passes=False
        ),
        out_type=jax.ShapeDtypeStruct((H, DT, BT * 8, 128), jnp.float32),
        scratch_types=[
            pltpu.VMEM((2, BW, 8, 128), jnp.int32),
            pltpu.VMEM((2, ROWS, D), jnp.float32),
            pltpu.VMEM((2 * DT * BW * 8, 128), jnp.float32),
            pltpu.SemaphoreType.DMA,
            pltpu.SemaphoreType.DMA,
            pltpu.SemaphoreType.DMA,
        ],
    )
    def emb(table_hbm, idx_hbm, out_hbm, idx_v, src_v, dst_v,
            sem_idx, sem_g, sem_o):
        wid = lax.axis_index("s") * NC + lax.axis_index("c")
        bt0 = wid * BW

        def idx_start(ht, islot):
            pltpu.async_copy(
                idx_hbm.at[ht].at[pl.ds(bt0, BW)], idx_v.at[islot], sem_idx
            )

        def idx_wait(islot):
            pltpu.make_async_copy(
                idx_hbm.at[0].at[pl.ds(bt0, BW)], idx_v.at[islot], sem_idx
            ).wait()

        def fire_gathers(step, sslot):
            hs = step % 8
            islot = (step // 8) % 2
            for btj in range(BW):
                pltpu.async_copy(
                    table_hbm.at[idx_v.at[islot].at[btj].at[hs]],
                    src_v.at[sslot].at[pl.ds(btj * 128, 128)],
                    sem_g,
                )

        def drain_gathers(sslot):
            pltpu.make_async_copy(
                table_hbm.at[idx_v.at[0].at[0].at[0]],
                src_v.at[sslot],
                sem_g,
            ).wait()

        def store_start(h, dslot):
            for dt in range(DT):
                pltpu.async_copy(
                    dst_v.at[pl.ds((dslot * DT + dt) * BW * 8, BW * 8)],
                    out_hbm.at[h].at[dt].at[pl.ds(bt0 * 8, BW * 8)],
                    sem_o,
                )

        def store_wait(dslot):
            pltpu.make_async_copy(
                dst_v.at[pl.ds(dslot * DT * BW * 8, DT * BW * 8)],
                out_hbm.at[0].at[0].at[pl.ds(0, DT * BW * 8)],
                sem_o,
            ).wait()

        lane = jnp.arange(_LANES, dtype=jnp.int32)
        r_const = (lane // 8) * (BW * 8) + (lane % 8)

        def transpose(b):
            src = src_v.at[b]

            @plsc.parallel_loop(0, ROWS, unroll=16)
            def _(i):
                btj = i // 128
                rloc = i - btj * 128
                bl_vec = jnp.zeros((_LANES,), jnp.int32) + rloc
                for half in range(2):
                    vec = src[i, pl.ds(half * _LANES, _LANES)]
                    row_vec = r_const + (
                        (b * DT + half * 2) * (BW * 8) + btj * 8
                    )
                    plsc.store_scatter(dst_v, [row_vec, bl_vec], vec)

        idx_start(0, 0)
        idx_wait(0)
        fire_gathers(0, 0)

        def outer(o, _):
            for b in range(2):
                step = o * 2 + b

                @pl.when(jnp.logical_and(step % 8 == 0, step + 8 < n_steps))
                def _():
                    idx_start(step // 8 + 1, (step // 8 + 1) % 2)

                drain_gathers(b)

                @pl.when(step + 1 < n_steps)
                def _():
                    @pl.when((step + 1) % 8 == 0)
                    def _():
                        idx_wait(((step + 1) // 8) % 2)

                    fire_gathers(step + 1, 1 - b)

                @pl.when(step >= 2)
                def _():
                    store_wait(b)

                transpose(b)
                store_start(step, b)
            return 0

        lax.fori_loop(0, n_steps // 2, outer, 0)
        for b in range(2):
            store_wait(b)

    return emb


def kernel(y, E):
    Bt, H = y.shape
    V, D = E.shape
    HT, BT, DT = H // 8, Bt // 128, D // 8
    y4 = y.T.reshape(HT, 8, BT, 128).transpose(0, 2, 1, 3)
    o4 = _build(V, D, Bt, H)(E, y4)
    o5 = o4.reshape(H, DT, BT, 8, 128)
    return o5.transpose(2, 4, 0, 1, 3).reshape(Bt, H, D)
